# initial kernel scaffold (unmeasured)
import jax
import jax.numpy as jnp
from jax import lax
from jax.experimental import pallas as pl
from jax.experimental.pallas import tpu as pltpu

T = 2048
V_HALF = 8192
R = 256
N_CHUNKS = T // R


def kernel(x, W):
    logits = jnp.dot(x, W, preferred_element_type=jnp.float32)
    return _exchange_softmax(logits)


def _exchange_softmax(logits):
    def body(logits_hbm, out_hbm, local_v, recv_v, out_v,
             load_sem, store_sem, send_sem, recv_sem, credit_sem):
        my_x = lax.axis_index("x")
        my_y = lax.axis_index("y")
        nbr = (my_x, 1 - my_y)

        barrier = pltpu.get_barrier_semaphore()
        pl.semaphore_signal(barrier, inc=1, device_id=nbr,
                            device_id_type=pl.DeviceIdType.MESH)
        pl.semaphore_wait(barrier, 1)

        for c in range(N_CHUNKS):
            load = pltpu.make_async_copy(
                logits_hbm.at[pl.ds(c * R, R), :], local_v, load_sem)
            load.start()
            load.wait()

            if c > 0:
                pl.semaphore_wait(credit_sem, 1)

            rdma = pltpu.make_async_remote_copy(
                src_ref=local_v, dst_ref=recv_v,
                send_sem=send_sem, recv_sem=recv_sem,
                device_id=nbr, device_id_type=pl.DeviceIdType.MESH)
            rdma.start()
            rdma.wait()

            loc = local_v[...]
            rem = recv_v[...]
            m = jnp.maximum(jnp.max(loc, axis=1, keepdims=True),
                            jnp.max(rem, axis=1, keepdims=True))
            el = jnp.exp(loc - m)
            er = jnp.exp(rem - m)
            denom = (jnp.sum(el, axis=1, keepdims=True)
                     + jnp.sum(er, axis=1, keepdims=True))
            out_v[:, pl.ds(my_y * V_HALF, V_HALF)] = el / denom
            out_v[:, pl.ds((1 - my_y) * V_HALF, V_HALF)] = er / denom

            pl.semaphore_signal(credit_sem, inc=1, device_id=nbr,
                                device_id_type=pl.DeviceIdType.MESH)

            store = pltpu.make_async_copy(
                out_v, out_hbm.at[pl.ds(c * R, R), :], store_sem)
            store.start()
            store.wait()

        pl.semaphore_wait(credit_sem, 1)

    return pl.pallas_call(
        body,
        out_shape=jax.ShapeDtypeStruct((T, 2 * V_HALF), jnp.float32),
        in_specs=[pl.BlockSpec(memory_space=pltpu.ANY)],
        out_specs=pl.BlockSpec(memory_space=pltpu.ANY),
        scratch_shapes=[
            pltpu.VMEM((R, V_HALF), jnp.float32),
            pltpu.VMEM((R, V_HALF), jnp.float32),
            pltpu.VMEM((R, 2 * V_HALF), jnp.float32),
            pltpu.SemaphoreType.DMA,
            pltpu.SemaphoreType.DMA,
            pltpu.SemaphoreType.DMA,
            pltpu.SemaphoreType.DMA,
            pltpu.SemaphoreType.REGULAR,
        ],
        compiler_params=pltpu.CompilerParams(collective_id=0),
    )(logits)


# baseline (device time: 1128507 ns/iter reference)
import jax
import jax.numpy as jnp
from jax import lax
from jax.experimental import pallas as pl
from jax.experimental.pallas import tpu as pltpu

T = 2048
V_HALF = 8192
R = 256
N_CHUNKS = T // R


def kernel(x, W):
    logits = jnp.dot(x, W, preferred_element_type=jnp.float32)
    return _exchange_softmax(logits)


def _exchange_softmax(logits):
    def body(logits_hbm, out_hbm, local_v, recv_v, out_v,
             load_sem, store_sem, send_sem, recv_sem, credit_sem):
        my_x = lax.axis_index("x")
        my_y = lax.axis_index("y")
        nbr = (my_x, 1 - my_y)

        barrier = pltpu.get_barrier_semaphore()
        pl.semaphore_signal(barrier, inc=1, device_id=nbr,
                            device_id_type=pl.DeviceIdType.MESH)
        pl.semaphore_wait(barrier, 1)

        for c in range(N_CHUNKS):
            load = pltpu.make_async_copy(
                logits_hbm.at[pl.ds(c * R, R), :], local_v, load_sem)
            load.start()
            load.wait()

            if c > 0:
                pl.semaphore_wait(credit_sem, 1)

            rdma = pltpu.make_async_remote_copy(
                src_ref=local_v, dst_ref=recv_v,
                send_sem=send_sem, recv_sem=recv_sem,
                device_id=nbr, device_id_type=pl.DeviceIdType.MESH)
            rdma.start()
            rdma.wait()

            loc = local_v[...]
            rem = recv_v[...]
            m = jnp.maximum(jnp.max(loc, axis=1, keepdims=True),
                            jnp.max(rem, axis=1, keepdims=True))
            el = jnp.exp(loc - m)
            er = jnp.exp(rem - m)
            denom = (jnp.sum(el, axis=1, keepdims=True)
                     + jnp.sum(er, axis=1, keepdims=True))
            out_v[:, pl.ds(my_y * V_HALF, V_HALF)] = el / denom
            out_v[:, pl.ds((1 - my_y) * V_HALF, V_HALF)] = er / denom

            pl.semaphore_signal(credit_sem, inc=1, device_id=nbr,
                                device_id_type=pl.DeviceIdType.MESH)

            store = pltpu.make_async_copy(
                out_v, out_hbm.at[pl.ds(c * R, R), :], store_sem)
            store.start()
            store.wait()

        pl.semaphore_wait(credit_sem, 1)

    return pl.pallas_call(
        body,
        out_shape=jax.ShapeDtypeStruct((T, 2 * V_HALF), jnp.float32),
        in_specs=[pl.BlockSpec(memory_space=pl.ANY)],
        out_specs=pl.BlockSpec(memory_space=pl.ANY),
        scratch_shapes=[
            pltpu.VMEM((R, V_HALF), jnp.float32),
            pltpu.VMEM((R, V_HALF), jnp.float32),
            pltpu.VMEM((R, 2 * V_HALF), jnp.float32),
            pltpu.SemaphoreType.DMA,
            pltpu.SemaphoreType.DMA,
            pltpu.SemaphoreType.DMA,
            pltpu.SemaphoreType.DMA,
            pltpu.SemaphoreType.REGULAR,
        ],
        compiler_params=pltpu.CompilerParams(
            collective_id=0, vmem_limit_bytes=100 * 1024 * 1024),
    )(logits)


# device time: 1012084 ns/iter; 1.1150x vs baseline; 1.1150x over previous
import jax
import jax.numpy as jnp
from jax import lax
from jax.experimental import pallas as pl
from jax.experimental.pallas import tpu as pltpu

T = 2048
V_HALF = 8192
R = 128
N_CHUNKS = T // R


def kernel(x, W):
    logits = jnp.dot(x, W, preferred_element_type=jnp.float32)
    return _exchange_softmax(logits)


def _exchange_softmax(logits):
    def body(logits_hbm, out_hbm, local_v, recv_v, out_v,
             load_sems, store_sems, send_sems, recv_sems, credit_sem):
        my_x = lax.axis_index("x")
        my_y = lax.axis_index("y")
        nbr = (my_x, 1 - my_y)

        def load(c):
            return pltpu.make_async_copy(
                logits_hbm.at[pl.ds(c * R, R), :], local_v.at[c % 2],
                load_sems.at[c % 2])

        def store(c):
            return pltpu.make_async_copy(
                out_v.at[c % 2], out_hbm.at[pl.ds(c * R, R), :],
                store_sems.at[c % 2])

        def rdma(c):
            s = c % 2
            return pltpu.make_async_remote_copy(
                src_ref=local_v.at[s], dst_ref=recv_v.at[s],
                send_sem=send_sems.at[s], recv_sem=recv_sems.at[s],
                device_id=nbr, device_id_type=pl.DeviceIdType.MESH)

        barrier = pltpu.get_barrier_semaphore()
        pl.semaphore_signal(barrier, inc=1, device_id=nbr,
                            device_id_type=pl.DeviceIdType.MESH)
        pl.semaphore_wait(barrier, 1)

        load(0).start()
        load(1).start()
        load(0).wait()
        rdma(0).start()

        for c in range(N_CHUNKS):
            s = c % 2
            if c + 1 < N_CHUNKS:
                load(c + 1).wait()
                if c + 1 >= 2:
                    pl.semaphore_wait(credit_sem, 1)
                rdma(c + 1).start()

            rdma(c).wait_recv()

            if c >= 2:
                store(c - 2).wait()

            loc = local_v[s]
            rem = recv_v[s]
            m = jnp.maximum(jnp.max(loc, axis=1, keepdims=True),
                            jnp.max(rem, axis=1, keepdims=True))
            el = jnp.exp(loc - m)
            er = jnp.exp(rem - m)
            denom = (jnp.sum(el, axis=1, keepdims=True)
                     + jnp.sum(er, axis=1, keepdims=True))
            out_v[s, :, pl.ds(my_y * V_HALF, V_HALF)] = el / denom
            out_v[s, :, pl.ds((1 - my_y) * V_HALF, V_HALF)] = er / denom

            pl.semaphore_signal(credit_sem, inc=1, device_id=nbr,
                                device_id_type=pl.DeviceIdType.MESH)
            store(c).start()

            rdma(c).wait_send()
            if c + 2 < N_CHUNKS:
                load(c + 2).start()

        store(N_CHUNKS - 2).wait()
        store(N_CHUNKS - 1).wait()
        pl.semaphore_wait(credit_sem, 2)

    return pl.pallas_call(
        body,
        out_shape=jax.ShapeDtypeStruct((T, 2 * V_HALF), jnp.float32),
        in_specs=[pl.BlockSpec(memory_space=pl.ANY)],
        out_specs=pl.BlockSpec(memory_space=pl.ANY),
        scratch_shapes=[
            pltpu.VMEM((2, R, V_HALF), jnp.float32),
            pltpu.VMEM((2, R, V_HALF), jnp.float32),
            pltpu.VMEM((2, R, 2 * V_HALF), jnp.float32),
            pltpu.SemaphoreType.DMA((2,)),
            pltpu.SemaphoreType.DMA((2,)),
            pltpu.SemaphoreType.DMA((2,)),
            pltpu.SemaphoreType.DMA((2,)),
            pltpu.SemaphoreType.REGULAR,
        ],
        compiler_params=pltpu.CompilerParams(
            collective_id=0, vmem_limit_bytes=63 * 1024 * 1024),
    )(logits)
